# Initial kernel scaffold; baseline (speedup 1.0000x reference)
import jax
import jax.numpy as jnp
from jax.experimental import pallas as pl


def kernel(x, edge_index, edge_values, W):
    raise NotImplementedError("write your pallas kernel here")


# speedup vs baseline: 4.6599x; 4.6599x over previous
import functools

import jax
import jax.numpy as jnp
from jax import lax
from jax.experimental import pallas as pl
from jax.experimental.pallas import tpu as pltpu
from jax.experimental.pallas import tpu_sc as plsc

N = 10000
E = 320000
D = 128

NC = 2
NS = 16
NW = NC * NS
LANES = 16

EPW = E // NW
CHUNK = 80
NCHUNK = EPW // CHUNK

WB_PT = 632
WB_LAST = N - 15 * WB_PT
ZROWS = 128


def _sc_spmm(x, edge_flat, edge_values):
    mesh = plsc.VectorSubcoreMesh(core_axis_name="c", subcore_axis_name="s")

    @functools.partial(
        pl.kernel,
        out_type=jax.ShapeDtypeStruct((NC * N, D), jnp.float32),
        mesh=mesh,
        scratch_types=[
            pltpu.VMEM((1, CHUNK), jnp.int32),
            pltpu.VMEM((1, CHUNK), jnp.int32),
            pltpu.VMEM((CHUNK,), jnp.float32),
            pltpu.VMEM((CHUNK, D), jnp.float32),
            pltpu.VMEM((ZROWS, D), jnp.float32),
            pltpu.VMEM_SHARED((N, D), jnp.float32),
            pltpu.SemaphoreType.DMA,
        ],
    )
    def spmm(x_hbm, ei_hbm, ev_hbm, out_hbm, sidx, didx, vals, rows, zblk,
             acc, sem):
        c = lax.axis_index("c")
        s = lax.axis_index("s")
        wid = s * NC + c

        zero = jnp.zeros((LANES,), jnp.float32)

        def zrow(i, _):
            for j in range(D // LANES):
                zblk[i, pl.ds(j * LANES, LANES)] = zero
            return 0

        lax.fori_loop(0, ZROWS, zrow, 0)

        def zfill(base_row, tail):
            for k in range(4):
                pltpu.sync_copy(
                    zblk, acc.at[pl.ds(base_row + k * ZROWS, ZROWS)])
            pltpu.sync_copy(zblk.at[pl.ds(0, tail)],
                            acc.at[pl.ds(base_row + 4 * ZROWS, tail)])

        @pl.when(s < NS - 1)
        def _():
            zfill(s * WB_PT, WB_PT - 4 * ZROWS)

        @pl.when(s == NS - 1)
        def _():
            zfill((NS - 1) * WB_PT, WB_LAST - 4 * ZROWS)

        plsc.subcore_barrier()

        base = wid * EPW

        def chunk(i, _):
            off = base + i * CHUNK
            pltpu.sync_copy(ei_hbm.at[pl.ds(off, CHUNK)], didx.at[0])
            pltpu.sync_copy(ei_hbm.at[pl.ds(E + off, CHUNK)], sidx.at[0])
            pltpu.sync_copy(ev_hbm.at[pl.ds(off, CHUNK)], vals)
            pltpu.async_copy(x_hbm.at[sidx.at[0]], rows, sem).wait()

            def scale(g, _):
                vv = vals[pl.ds(g * LANES, LANES)]
                for t in range(LANES):
                    vsp = vv.at[jnp.full((LANES,), t, jnp.int32)].get(
                        mode="promise_in_bounds")
                    e = g * LANES + t
                    for j in range(D // LANES):
                        sl = pl.ds(j * LANES, LANES)
                        rows[e, sl] = rows[e, sl] * vsp
                return 0

            lax.fori_loop(0, CHUNK // LANES, scale, 0)
            pltpu.sync_copy(rows, acc.at[didx.at[0]], add=True)
            return 0

        lax.fori_loop(0, NCHUNK, chunk, 0)

        plsc.subcore_barrier()

        @pl.when(s < NS - 1)
        def _():
            pltpu.sync_copy(
                acc.at[pl.ds(s * WB_PT, WB_PT)],
                out_hbm.at[pl.ds(c * N + s * WB_PT, WB_PT)])

        @pl.when(s == NS - 1)
        def _():
            pltpu.sync_copy(
                acc.at[pl.ds((NS - 1) * WB_PT, WB_LAST)],
                out_hbm.at[pl.ds(c * N + (NS - 1) * WB_PT, WB_LAST)])

    return spmm(x, edge_flat, edge_values)


BLK = 1000


def _tc_finish(partials, W):
    def body(p0_ref, p1_ref, w_ref, o_ref):
        ssum = p0_ref[...] + p1_ref[...]
        o_ref[...] = lax.dot_general(
            ssum, w_ref[...], (((1,), (1,)), ((), ())),
            preferred_element_type=jnp.float32)

    return pl.pallas_call(
        body,
        grid=(N // BLK,),
        in_specs=[
            pl.BlockSpec((BLK, D), lambda i: (i, 0)),
            pl.BlockSpec((BLK, D), lambda i: (N // BLK + i, 0)),
            pl.BlockSpec((D, D), lambda i: (0, 0)),
        ],
        out_specs=pl.BlockSpec((BLK, D), lambda i: (i, 0)),
        out_shape=jax.ShapeDtypeStruct((N, D), jnp.float32),
    )(partials, partials, W)


def kernel(x, edge_index, edge_values, W):
    partials = _sc_spmm(x, edge_index.reshape(-1), edge_values)
    return _tc_finish(partials, W)
